# Initial kernel scaffold; baseline (speedup 1.0000x reference)
import functools

import jax
import jax.numpy as jnp
from jax import lax
from jax.experimental import pallas as pl
from jax.experimental.pallas import tpu as pltpu
from jax.experimental.pallas import tpu_sc as plsc

_N_USERS = 50000
_N_ITEMS = 50000
_N_NODES = _N_USERS + _N_ITEMS
_D = 32
_E = 1600000

_NC = 2
_NS = 16
_L = 16

_HALF = _N_NODES // _NC
_ACC_ROWS = 50176
_ZPT = _ACC_ROWS // _NS
_ZROWS = 392
_CHUNK_G = 8
_CHUNK = _CHUNK_G * 128
_NCHUNKS = 98
_EPT = _CHUNK * _NCHUNKS
_E_PAD = _EPT * _NS
_WPT = _HALF // _NS
_WB = 625


def _layer_body(scale, emb_in, sum_in, src2, dst2, w2, emb_out, sum_out,
                acc, srcv, dstv, locv, wv, rows, bufa, bufb, zbuf, sem):
    c = lax.axis_index("c")
    s = lax.axis_index("s")
    zero = jnp.zeros((_L,), jnp.float32)

    @pl.loop(0, _ZROWS)
    def _zero_zbuf(r):
        zbuf[r, pl.ds(0, _L)] = zero
        zbuf[r, pl.ds(_L, _L)] = zero

    for k in range(_ZPT // _ZROWS):
        pltpu.sync_copy(zbuf, acc.at[pl.ds(s * _ZPT + k * _ZROWS, _ZROWS)])
    plsc.subcore_barrier()

    lo = c * _HALF
    trow0 = s * (_EPT // 128)

    @pl.loop(0, _NCHUNKS)
    def _chunk(t):
        row = trow0 + t * _CHUNK_G
        pltpu.sync_copy(src2.at[pl.ds(row, _CHUNK_G)], srcv)
        pltpu.sync_copy(dst2.at[pl.ds(row, _CHUNK_G)], dstv)
        pltpu.sync_copy(w2.at[pl.ds(row, _CHUNK_G)], wv)
        pltpu.async_copy(emb_in.at[srcv], rows, sem).wait()
        for g in range(_CHUNK_G):
            for h in range(128 // _L):
                d16 = dstv[g, pl.ds(h * _L, _L)]
                ok = (d16 >= lo) & (d16 < lo + _HALF)
                locv[g, pl.ds(h * _L, _L)] = jnp.where(ok, d16 - lo, _HALF)

        @pl.loop(0, _CHUNK, unroll=8)
        def _scale_rows(e):
            g = e >> 7
            col = e & 127
            wspl = jnp.full((_L,), wv[g, col], jnp.float32)
            rows[g, col, pl.ds(0, _L)] = rows[g, col, pl.ds(0, _L)] * wspl
            rows[g, col, pl.ds(_L, _L)] = rows[g, col, pl.ds(_L, _L)] * wspl

        pltpu.sync_copy(rows, acc.at[locv], add=True)

    plsc.subcore_barrier()

    for k in range(_WPT // _WB):
        lrow = s * _WPT + k * _WB
        grow = c * _HALF + s * _WPT + k * _WB
        pltpu.sync_copy(acc.at[pl.ds(lrow, _WB)], bufa)
        pltpu.sync_copy(sum_in.at[pl.ds(grow, _WB)], bufb)

        @pl.loop(0, _WB)
        def _accumulate(r):
            for half in (0, _L):
                v = bufa[r, pl.ds(half, _L)] + bufb[r, pl.ds(half, _L)]
                if scale != 1.0:
                    v = v * scale
                bufb[r, pl.ds(half, _L)] = v

        pltpu.sync_copy(bufa, emb_out.at[pl.ds(grow, _WB)])
        pltpu.sync_copy(bufb, sum_out.at[pl.ds(grow, _WB)])


@functools.lru_cache(maxsize=None)
def _make_layer(scale):
    mesh = plsc.VectorSubcoreMesh(
        core_axis_name="c", subcore_axis_name="s",
        num_cores=_NC, num_subcores=_NS)
    out_type = (jax.ShapeDtypeStruct((_N_NODES, _D), jnp.float32),
                jax.ShapeDtypeStruct((_N_NODES, _D), jnp.float32))
    scratch = [
        pltpu.VMEM_SHARED((_ACC_ROWS, _D), jnp.float32),
        pltpu.VMEM((_CHUNK_G, 128), jnp.int32),
        pltpu.VMEM((_CHUNK_G, 128), jnp.int32),
        pltpu.VMEM((_CHUNK_G, 128), jnp.int32),
        pltpu.VMEM((_CHUNK_G, 128), jnp.float32),
        pltpu.VMEM((_CHUNK_G, 128, _D), jnp.float32),
        pltpu.VMEM((_WB, _D), jnp.float32),
        pltpu.VMEM((_WB, _D), jnp.float32),
        pltpu.VMEM((_ZROWS, _D), jnp.float32),
        pltpu.SemaphoreType.DMA,
    ]
    return pl.kernel(functools.partial(_layer_body, scale),
                     out_type=out_type, mesh=mesh, scratch_types=scratch)


def kernel(edge_index, edge_weight, user_emb_w, item_emb_w):
    src = edge_index[0].astype(jnp.int32)
    dst = edge_index[1].astype(jnp.int32)
    w = edge_weight.astype(jnp.float32)
    pad = _E_PAD - _E
    src2 = jnp.concatenate([src, jnp.zeros((pad,), jnp.int32)]).reshape(-1, 128)
    dst2 = jnp.concatenate([dst, jnp.zeros((pad,), jnp.int32)]).reshape(-1, 128)
    w2 = jnp.concatenate([w, jnp.zeros((pad,), jnp.float32)]).reshape(-1, 128)
    emb = jnp.concatenate([user_emb_w, item_emb_w], axis=0)
    acc = emb
    for layer in range(3):
        emb, acc = _make_layer(0.25 if layer == 2 else 1.0)(
            emb, acc, src2, dst2, w2)
    return acc[:_N_USERS], acc[_N_USERS:]


# speedup vs baseline: 7.0467x; 7.0467x over previous
import functools

import jax
import jax.numpy as jnp
import numpy as np
from jax import lax
from jax.experimental import pallas as pl
from jax.experimental.pallas import tpu as pltpu
from jax.experimental.pallas import tpu_sc as plsc

_N_USERS = 50000
_N_ITEMS = 50000
_N_NODES = _N_USERS + _N_ITEMS
_D = 32
_E = 1600000

_NC = 2
_NS = 16
_L = 16

_HALF = _N_NODES // _NC
_PAD_HALF = 50176
_PAD_NODES = _PAD_HALF * _NC
_ACC_ROWS = _PAD_HALF
_RPT = _PAD_HALF // _NS
_CHUNK_G = 4
_CHUNK = _CHUNK_G * 128
_NCHUNKS = 196
_EPT = _CHUNK * _NCHUNKS
_E_PAD = _EPT * _NS
_WB = 112


def _layer_body(scale, emb_in, sum_in, src2, dst2, w2, emb_out, sum_out,
                acc, srcv, dstv, locv, wv, rows, sem):
    c = lax.axis_index("c")
    s = lax.axis_index("s")
    zero = jnp.zeros((_L,), jnp.float32)

    @pl.loop(np.int32(0), np.int32(_WB))
    def _zero_rows(r):
        rows[r, pl.ds(0, _L)] = zero
        rows[r, pl.ds(_L, _L)] = zero

    for k in range(_RPT // _WB):
        pltpu.sync_copy(rows.at[pl.ds(0, _WB)],
                        acc.at[pl.ds(s * np.int32(_RPT) + np.int32(k * _WB),
                                     _WB)])
    plsc.subcore_barrier()

    lo = c * np.int32(_HALF)
    trow0 = s * np.int32(_EPT // 128)

    @pl.loop(np.int32(0), np.int32(_NCHUNKS))
    def _chunk(t):
        row = trow0 + t * np.int32(_CHUNK_G)
        pltpu.sync_copy(src2.at[pl.ds(row, _CHUNK_G)], srcv)
        pltpu.sync_copy(dst2.at[pl.ds(row, _CHUNK_G)], dstv)
        pltpu.sync_copy(w2.at[pl.ds(row, _CHUNK_G)], wv)
        for g in range(_CHUNK_G):
            for h in range(128 // _L):
                s16 = srcv[g, pl.ds(h * _L, _L)]
                srcv[g, pl.ds(h * _L, _L)] = jnp.where(
                    s16 >= np.int32(_N_USERS),
                    s16 + np.int32(_PAD_HALF - _HALF), s16)
        descs = [pltpu.async_copy(emb_in.at[srcv.at[g]],
                                  rows.at[pl.ds(g * 128, 128)], sem)
                 for g in range(_CHUNK_G)]
        for g in range(_CHUNK_G):
            for h in range(128 // _L):
                d16 = dstv[g, pl.ds(h * _L, _L)]
                ok = (d16 >= lo) & (d16 < lo + np.int32(_HALF))
                locv[g, pl.ds(h * _L, _L)] = jnp.where(
                    ok, d16 - lo, np.int32(_HALF))
        for d in descs:
            d.wait()

        @pl.loop(np.int32(0), np.int32(_CHUNK // _L))
        def _scale_rows(q):
            g = q >> np.int32(3)
            h = q & np.int32(7)
            w16 = wv[g, pl.ds(h * np.int32(_L), _L)]
            e0 = q * np.int32(_L)
            for j in range(_L):
                e = e0 + np.int32(j)
                wspl = jnp.full((_L,), w16[j], jnp.float32)
                rows[e, pl.ds(0, _L)] = rows[e, pl.ds(0, _L)] * wspl
                rows[e, pl.ds(_L, _L)] = rows[e, pl.ds(_L, _L)] * wspl

        for g in range(_CHUNK_G):
            pltpu.sync_copy(rows.at[pl.ds(g * 128, 128)],
                            acc.at[locv.at[g]], add=True)

    plsc.subcore_barrier()

    for k in range(_RPT // _WB):
        lrow = s * np.int32(_RPT) + np.int32(k * _WB)
        grow = c * np.int32(_PAD_HALF) + lrow
        pltpu.sync_copy(acc.at[pl.ds(lrow, _WB)], rows.at[pl.ds(0, _WB)])
        pltpu.sync_copy(sum_in.at[pl.ds(grow, _WB)],
                        rows.at[pl.ds(_WB, _WB)])

        @pl.loop(np.int32(0), np.int32(_WB))
        def _accumulate(r):
            r2 = r + np.int32(_WB)
            for half in (0, _L):
                v = rows[r, pl.ds(half, _L)] + rows[r2, pl.ds(half, _L)]
                if scale != 1.0:
                    v = v * np.float32(scale)
                rows[r2, pl.ds(half, _L)] = v

        pltpu.sync_copy(rows.at[pl.ds(0, _WB)], emb_out.at[pl.ds(grow, _WB)])
        pltpu.sync_copy(rows.at[pl.ds(_WB, _WB)],
                        sum_out.at[pl.ds(grow, _WB)])


@functools.lru_cache(maxsize=None)
def _make_layer(scale):
    mesh = plsc.VectorSubcoreMesh(
        core_axis_name="c", subcore_axis_name="s",
        num_cores=_NC, num_subcores=_NS)
    out_type = (jax.ShapeDtypeStruct((_PAD_NODES, _D), jnp.float32),
                jax.ShapeDtypeStruct((_PAD_NODES, _D), jnp.float32))
    scratch = [
        pltpu.VMEM_SHARED((_ACC_ROWS, _D), jnp.float32),
        pltpu.VMEM((_CHUNK_G, 128), jnp.int32),
        pltpu.VMEM((_CHUNK_G, 128), jnp.int32),
        pltpu.VMEM((_CHUNK_G, 128), jnp.int32),
        pltpu.VMEM((_CHUNK_G, 128), jnp.float32),
        pltpu.VMEM((_CHUNK, _D), jnp.float32),
        pltpu.SemaphoreType.DMA,
    ]
    return pl.kernel(functools.partial(_layer_body, scale),
                     out_type=out_type, mesh=mesh, scratch_types=scratch,
                     compiler_params=pltpu.CompilerParams(
                         use_tc_tiling_on_sc=False))


def kernel(edge_index, edge_weight, user_emb_w, item_emb_w):
    with jax.enable_x64(False):
        return _kernel_x32(edge_index.astype(jnp.int32),
                           edge_weight.astype(jnp.float32),
                           user_emb_w.astype(jnp.float32),
                           item_emb_w.astype(jnp.float32))


def _kernel_x32(edge_index, edge_weight, user_emb_w, item_emb_w):
    src = edge_index[0]
    dst = edge_index[1]
    w = edge_weight
    pad = _E_PAD - _E
    src2 = jnp.concatenate([src, jnp.zeros((pad,), jnp.int32)]).reshape(-1, 128)
    dst2 = jnp.concatenate([dst, jnp.zeros((pad,), jnp.int32)]).reshape(-1, 128)
    w2 = jnp.concatenate([w, jnp.zeros((pad,), jnp.float32)]).reshape(-1, 128)
    zpad = jnp.zeros((_PAD_HALF - _HALF, _D), jnp.float32)
    emb = jnp.concatenate([user_emb_w, zpad, item_emb_w, zpad], axis=0)
    acc = emb
    for layer in range(3):
        emb, acc = _make_layer(0.25 if layer == 2 else 1.0)(
            emb, acc, src2, dst2, w2)
    return (acc[:_N_USERS],
            acc[_PAD_HALF:_PAD_HALF + _N_ITEMS])


# speedup vs baseline: 8.5965x; 1.2199x over previous
import functools

import jax
import jax.numpy as jnp
import numpy as np
from jax import lax
from jax.experimental import pallas as pl
from jax.experimental.pallas import tpu as pltpu
from jax.experimental.pallas import tpu_sc as plsc

_N_USERS = 50000
_N_ITEMS = 50000
_N_NODES = _N_USERS + _N_ITEMS
_D = 32
_E = 1600000

_NC = 2
_NS = 16
_L = 16
_DH = _D // _NC

_PAD_NODES = 100352
_RPT = _PAD_NODES // _NS
_CHUNK_G = 4
_CHUNK = _CHUNK_G * 128
_NCHUNKS = 196
_EPT = _CHUNK * _NCHUNKS
_E_PAD = _EPT * _NS
_WB = 224


def _layer_body(scale, emb_in, sum_in, src2, dst2, w2, emb_out, sum_out,
                acc, srcv, dstv, wv, rows, sem):
    c = lax.axis_index("c")
    s = lax.axis_index("s")
    zero = jnp.zeros((_L,), jnp.float32)

    @pl.loop(np.int32(0), np.int32(_WB))
    def _zero_rows(r):
        rows[r] = zero

    for k in range(_RPT // _WB):
        pltpu.sync_copy(rows.at[pl.ds(0, _WB)],
                        acc.at[pl.ds(s * np.int32(_RPT) + np.int32(k * _WB),
                                     _WB)])
    plsc.subcore_barrier()

    trow0 = s * np.int32(_EPT // 128)

    @pl.loop(np.int32(0), np.int32(_NCHUNKS))
    def _chunk(t):
        row = trow0 + t * np.int32(_CHUNK_G)
        pltpu.sync_copy(src2.at[pl.ds(row, _CHUNK_G)], srcv)
        pltpu.sync_copy(dst2.at[pl.ds(row, _CHUNK_G)], dstv)
        pltpu.sync_copy(w2.at[pl.ds(row, _CHUNK_G)], wv)
        descs = [pltpu.async_copy(emb_in.at[c].at[srcv.at[g]],
                                  rows.at[pl.ds(g * 128, 128)], sem)
                 for g in range(_CHUNK_G)]
        for d in descs:
            d.wait()

        @pl.loop(np.int32(0), np.int32(_CHUNK // _L))
        def _scale_rows(q):
            g = q >> np.int32(3)
            h = q & np.int32(7)
            w16 = wv[g, pl.ds(h * np.int32(_L), _L)]
            e0 = q * np.int32(_L)
            for j in range(_L):
                e = e0 + np.int32(j)
                wspl = jnp.full((_L,), w16[j], jnp.float32)
                rows[e] = rows[e] * wspl

        for g in range(_CHUNK_G):
            pltpu.sync_copy(rows.at[pl.ds(g * 128, 128)],
                            acc.at[dstv.at[g]], add=True)

    plsc.subcore_barrier()

    for k in range(_RPT // _WB):
        lrow = s * np.int32(_RPT) + np.int32(k * _WB)
        pltpu.sync_copy(acc.at[pl.ds(lrow, _WB)], rows.at[pl.ds(0, _WB)])
        pltpu.sync_copy(sum_in.at[c].at[pl.ds(lrow, _WB)],
                        rows.at[pl.ds(_WB, _WB)])

        @pl.loop(np.int32(0), np.int32(_WB))
        def _accumulate(r):
            v = rows[r] + rows[r + np.int32(_WB)]
            if scale != 1.0:
                v = v * np.float32(scale)
            rows[r + np.int32(_WB)] = v

        pltpu.sync_copy(rows.at[pl.ds(0, _WB)],
                        emb_out.at[c].at[pl.ds(lrow, _WB)])
        pltpu.sync_copy(rows.at[pl.ds(_WB, _WB)],
                        sum_out.at[c].at[pl.ds(lrow, _WB)])


@functools.lru_cache(maxsize=None)
def _make_layer(scale):
    mesh = plsc.VectorSubcoreMesh(
        core_axis_name="c", subcore_axis_name="s",
        num_cores=_NC, num_subcores=_NS)
    out_type = (jax.ShapeDtypeStruct((_NC, _PAD_NODES, _DH), jnp.float32),
                jax.ShapeDtypeStruct((_NC, _PAD_NODES, _DH), jnp.float32))
    scratch = [
        pltpu.VMEM_SHARED((_PAD_NODES, _DH), jnp.float32),
        pltpu.VMEM((_CHUNK_G, 128), jnp.int32),
        pltpu.VMEM((_CHUNK_G, 128), jnp.int32),
        pltpu.VMEM((_CHUNK_G, 128), jnp.float32),
        pltpu.VMEM((_CHUNK, _DH), jnp.float32),
        pltpu.SemaphoreType.DMA,
    ]
    return pl.kernel(functools.partial(_layer_body, scale),
                     out_type=out_type, mesh=mesh, scratch_types=scratch,
                     compiler_params=pltpu.CompilerParams(
                         use_tc_tiling_on_sc=False))


def kernel(edge_index, edge_weight, user_emb_w, item_emb_w):
    with jax.enable_x64(False):
        return _kernel_x32(edge_index.astype(jnp.int32),
                           edge_weight.astype(jnp.float32),
                           user_emb_w.astype(jnp.float32),
                           item_emb_w.astype(jnp.float32))


def _kernel_x32(edge_index, edge_weight, user_emb_w, item_emb_w):
    src = edge_index[0]
    dst = edge_index[1]
    w = edge_weight
    pad = _E_PAD - _E
    src2 = jnp.concatenate([src, jnp.zeros((pad,), jnp.int32)]).reshape(-1, 128)
    dst2 = jnp.concatenate([dst, jnp.zeros((pad,), jnp.int32)]).reshape(-1, 128)
    w2 = jnp.concatenate([w, jnp.zeros((pad,), jnp.float32)]).reshape(-1, 128)
    emb0 = jnp.concatenate([user_emb_w, item_emb_w], axis=0)
    emb0 = jnp.concatenate(
        [emb0, jnp.zeros((_PAD_NODES - _N_NODES, _D), jnp.float32)], axis=0)
    emb = jnp.stack([emb0[:, :_DH], emb0[:, _DH:]], axis=0)
    acc = emb
    for layer in range(3):
        emb, acc = _make_layer(0.25 if layer == 2 else 1.0)(
            emb, acc, src2, dst2, w2)
    final = jnp.concatenate([acc[0, :_N_NODES], acc[1, :_N_NODES]], axis=1)
    return final[:_N_USERS], final[_N_USERS:]


# speedup vs baseline: 8.8209x; 1.0261x over previous
import functools

import jax
import jax.numpy as jnp
import numpy as np
from jax import lax
from jax.experimental import pallas as pl
from jax.experimental.pallas import tpu as pltpu
from jax.experimental.pallas import tpu_sc as plsc

_N_USERS = 50000
_N_ITEMS = 50000
_N_NODES = _N_USERS + _N_ITEMS
_D = 32
_E = 1600000

_NC = 2
_NS = 16
_L = 16
_DH = _D // _NC

_PAD_NODES = 100352
_RPT = _PAD_NODES // _NS
_CHUNK_G = 4
_CHUNK = _CHUNK_G * 128
_NCHUNKS = 196
_EPT = _CHUNK * _NCHUNKS
_E_PAD = _EPT * _NS
_WB = 224

_LANE = [np.full((_L,), j, np.int32) for j in range(_L)]


def _layer_body(scale, emb_in, sum_in, src2, dst2, w2, emb_out, sum_out,
                acc, srcv, dstv, wv, rows, sem):
    c = lax.axis_index("c")
    s = lax.axis_index("s")
    zero = jnp.zeros((_L,), jnp.float32)

    @pl.loop(np.int32(0), np.int32(_WB))
    def _zero_rows(r):
        rows[r] = zero

    for k in range(_RPT // _WB):
        pltpu.sync_copy(rows.at[pl.ds(0, _WB)],
                        acc.at[pl.ds(s * np.int32(_RPT) + np.int32(k * _WB),
                                     _WB)])
    plsc.subcore_barrier()

    trow0 = s * np.int32(_EPT // 128)

    @pl.loop(np.int32(0), np.int32(_NCHUNKS))
    def _chunk(t):
        row = trow0 + t * np.int32(_CHUNK_G)
        pltpu.sync_copy(src2.at[pl.ds(row, _CHUNK_G)], srcv)
        pltpu.sync_copy(dst2.at[pl.ds(row, _CHUNK_G)], dstv)
        pltpu.sync_copy(w2.at[pl.ds(row, _CHUNK_G)], wv)
        descs = [pltpu.async_copy(emb_in.at[c].at[srcv.at[g]],
                                  rows.at[pl.ds(g * 128, 128)], sem)
                 for g in range(_CHUNK_G)]
        for d in descs:
            d.wait()

        @plsc.parallel_loop(np.int32(0), np.int32(_CHUNK // _L),
                            unroll=2)
        def _scale_rows(q):
            g = q >> np.int32(3)
            h = q & np.int32(7)
            w16 = wv[g, pl.ds(h * np.int32(_L), _L)]
            e0 = q * np.int32(_L)
            for j in range(_L):
                e = e0 + np.int32(j)
                wspl = jnp.full((_L,), w16[j], jnp.float32)
                rows[e] = rows[e] * wspl

        for g in range(_CHUNK_G):
            pltpu.sync_copy(rows.at[pl.ds(g * 128, 128)],
                            acc.at[dstv.at[g]], add=True)

    plsc.subcore_barrier()

    for k in range(_RPT // _WB):
        lrow = s * np.int32(_RPT) + np.int32(k * _WB)
        pltpu.sync_copy(acc.at[pl.ds(lrow, _WB)], rows.at[pl.ds(0, _WB)])
        pltpu.sync_copy(sum_in.at[c].at[pl.ds(lrow, _WB)],
                        rows.at[pl.ds(_WB, _WB)])

        @pl.loop(np.int32(0), np.int32(_WB))
        def _accumulate(r):
            v = rows[r] + rows[r + np.int32(_WB)]
            if scale != 1.0:
                v = v * np.float32(scale)
            rows[r + np.int32(_WB)] = v

        pltpu.sync_copy(rows.at[pl.ds(0, _WB)],
                        emb_out.at[c].at[pl.ds(lrow, _WB)])
        pltpu.sync_copy(rows.at[pl.ds(_WB, _WB)],
                        sum_out.at[c].at[pl.ds(lrow, _WB)])


@functools.lru_cache(maxsize=None)
def _make_layer(scale):
    mesh = plsc.VectorSubcoreMesh(
        core_axis_name="c", subcore_axis_name="s",
        num_cores=_NC, num_subcores=_NS)
    out_type = (jax.ShapeDtypeStruct((_NC, _PAD_NODES, _DH), jnp.float32),
                jax.ShapeDtypeStruct((_NC, _PAD_NODES, _DH), jnp.float32))
    scratch = [
        pltpu.VMEM_SHARED((_PAD_NODES, _DH), jnp.float32),
        pltpu.VMEM((_CHUNK_G, 128), jnp.int32),
        pltpu.VMEM((_CHUNK_G, 128), jnp.int32),
        pltpu.VMEM((_CHUNK_G, 128), jnp.float32),
        pltpu.VMEM((_CHUNK, _DH), jnp.float32),
        pltpu.SemaphoreType.DMA,
    ]
    return pl.kernel(functools.partial(_layer_body, scale),
                     out_type=out_type, mesh=mesh, scratch_types=scratch,
                     compiler_params=pltpu.CompilerParams(
                         use_tc_tiling_on_sc=False))


def kernel(edge_index, edge_weight, user_emb_w, item_emb_w):
    with jax.enable_x64(False):
        return _kernel_x32(edge_index.astype(jnp.int32),
                           edge_weight.astype(jnp.float32),
                           user_emb_w.astype(jnp.float32),
                           item_emb_w.astype(jnp.float32))


def _kernel_x32(edge_index, edge_weight, user_emb_w, item_emb_w):
    src = edge_index[0]
    dst = edge_index[1]
    w = edge_weight
    pad = _E_PAD - _E
    src2 = jnp.concatenate([src, jnp.zeros((pad,), jnp.int32)]).reshape(-1, 128)
    dst2 = jnp.concatenate([dst, jnp.zeros((pad,), jnp.int32)]).reshape(-1, 128)
    w2 = jnp.concatenate([w, jnp.zeros((pad,), jnp.float32)]).reshape(-1, 128)
    emb0 = jnp.concatenate([user_emb_w, item_emb_w], axis=0)
    emb0 = jnp.concatenate(
        [emb0, jnp.zeros((_PAD_NODES - _N_NODES, _D), jnp.float32)], axis=0)
    emb = jnp.stack([emb0[:, :_DH], emb0[:, _DH:]], axis=0)
    acc = emb
    for layer in range(3):
        emb, acc = _make_layer(0.25 if layer == 2 else 1.0)(
            emb, acc, src2, dst2, w2)
    final = jnp.concatenate([acc[0, :_N_NODES], acc[1, :_N_NODES]], axis=1)
    return final[:_N_USERS], final[_N_USERS:]


# speedup vs baseline: 13.3563x; 1.5142x over previous
import functools

import jax
import jax.numpy as jnp
import numpy as np
from jax import lax
from jax.experimental import pallas as pl
from jax.experimental.pallas import tpu as pltpu
from jax.experimental.pallas import tpu_sc as plsc

_N_USERS = 50000
_N_ITEMS = 50000
_N_NODES = _N_USERS + _N_ITEMS
_D = 32
_E = 1600000

_NC = 2
_NS = 16
_L = 16
_DH = _D // _NC

_PAD_NODES = 100352
_RPT = _PAD_NODES // _NS
_CHUNK_G = 4
_CHUNK = _CHUNK_G * 128
_NCHUNKS = 198
_EPT = _CHUNK * _NCHUNKS
_E_PAD = _EPT * _NS
_NCHUNKS_TOT = _E_PAD // _CHUNK
_WB = 448


def _layer_body(scale, emb_in, sum_in, meta, emb_out, sum_out,
                acc, idxq, rowsq, sem_i, sem_g, sem_s):
    c = lax.axis_index("c")
    s = lax.axis_index("s")
    zero = jnp.zeros((_L,), jnp.float32)

    @pl.loop(np.int32(0), np.int32(_WB))
    def _zero_rows(r):
        rows_z = rowsq.at[r]
        rows_z[...] = zero

    for k in range(_RPT // _WB):
        pltpu.sync_copy(rowsq.at[pl.ds(0, _WB)],
                        acc.at[pl.ds(s * np.int32(_RPT) + np.int32(k * _WB),
                                     _WB)])
    plsc.subcore_barrier()

    cg0 = s * np.int32(_NCHUNKS)

    def fire_idx(t, q):
        pltpu.async_copy(meta.at[cg0 + t], idxq.at[q], sem_i.at[q])

    def fire_gathers(t, q):
        off = q * np.int32(_CHUNK)
        for g in range(_CHUNK_G):
            pltpu.async_copy(
                emb_in.at[c].at[idxq.at[q, 0, g]],
                rowsq.at[pl.ds(off + np.int32(g * 128), 128)],
                sem_g.at[q])

    fire_idx(np.int32(0), np.int32(0))
    fire_idx(np.int32(1), np.int32(1))
    pltpu.make_async_copy(meta.at[0], idxq.at[0], sem_i.at[0]).wait()
    fire_gathers(np.int32(0), np.int32(0))

    @pl.loop(np.int32(0), np.int32(_NCHUNKS))
    def _chunk(t):
        p = t & np.int32(1)
        pn = np.int32(1) - p
        poff = p * np.int32(_CHUNK)

        @pl.when(t <= np.int32(_NCHUNKS - 2))
        def _fire_next():
            pltpu.make_async_copy(meta.at[0], idxq.at[pn],
                                  sem_i.at[pn]).wait()
            fire_gathers(t + np.int32(1), pn)

        pltpu.make_async_copy(emb_in.at[c].at[pl.ds(0, _CHUNK)],
                              rowsq.at[pl.ds(poff, _CHUNK)],
                              sem_g.at[p]).wait()

        @plsc.parallel_loop(np.int32(0), np.int32(_CHUNK // _L), unroll=2)
        def _scale_rows(q):
            g = q >> np.int32(3)
            h = q & np.int32(7)
            w16 = plsc.bitcast(idxq[p, 2, g, pl.ds(h * np.int32(_L), _L)],
                               jnp.float32)
            e0 = poff + q * np.int32(_L)
            for j in range(_L):
                e = e0 + np.int32(j)
                wspl = jnp.full((_L,), w16[j], jnp.float32)
                rowsq[e] = rowsq[e] * wspl

        descs = [pltpu.async_copy(
                     rowsq.at[pl.ds(poff + np.int32(g * 128), 128)],
                     acc.at[idxq.at[p, 1, g]], sem_s, add=True)
                 for g in range(_CHUNK_G)]
        for d in descs:
            d.wait()

        @pl.when(t <= np.int32(_NCHUNKS - 3))
        def _prefetch_idx():
            fire_idx(t + np.int32(2), p)

    plsc.subcore_barrier()

    for k in range(_RPT // _WB):
        lrow = s * np.int32(_RPT) + np.int32(k * _WB)
        pltpu.sync_copy(acc.at[pl.ds(lrow, _WB)], rowsq.at[pl.ds(0, _WB)])
        pltpu.sync_copy(sum_in.at[c].at[pl.ds(lrow, _WB)],
                        rowsq.at[pl.ds(_CHUNK, _WB)])

        @pl.loop(np.int32(0), np.int32(_WB))
        def _accumulate(r):
            v = rowsq[r] + rowsq[r + np.int32(_CHUNK)]
            if scale != 1.0:
                v = v * np.float32(scale)
            rowsq[r + np.int32(_CHUNK)] = v

        pltpu.sync_copy(rowsq.at[pl.ds(0, _WB)],
                        emb_out.at[c].at[pl.ds(lrow, _WB)])
        pltpu.sync_copy(rowsq.at[pl.ds(_CHUNK, _WB)],
                        sum_out.at[c].at[pl.ds(lrow, _WB)])


@functools.lru_cache(maxsize=None)
def _make_layer(scale):
    mesh = plsc.VectorSubcoreMesh(
        core_axis_name="c", subcore_axis_name="s",
        num_cores=_NC, num_subcores=_NS)
    out_type = (jax.ShapeDtypeStruct((_NC, _PAD_NODES, _DH), jnp.float32),
                jax.ShapeDtypeStruct((_NC, _PAD_NODES, _DH), jnp.float32))
    scratch = [
        pltpu.VMEM_SHARED((_PAD_NODES, _DH), jnp.float32),
        pltpu.VMEM((2, 3, _CHUNK_G, 128), jnp.int32),
        pltpu.VMEM((2 * _CHUNK, _DH), jnp.float32),
        pltpu.SemaphoreType.DMA((2,)),
        pltpu.SemaphoreType.DMA((2,)),
        pltpu.SemaphoreType.DMA,
    ]
    return pl.kernel(functools.partial(_layer_body, scale),
                     out_type=out_type, mesh=mesh, scratch_types=scratch,
                     compiler_params=pltpu.CompilerParams(
                         use_tc_tiling_on_sc=False,
                         needs_layout_passes=False))


def kernel(edge_index, edge_weight, user_emb_w, item_emb_w):
    with jax.enable_x64(False):
        return _kernel_x32(edge_index.astype(jnp.int32),
                           edge_weight.astype(jnp.float32),
                           user_emb_w.astype(jnp.float32),
                           item_emb_w.astype(jnp.float32))


def _kernel_x32(edge_index, edge_weight, user_emb_w, item_emb_w):
    pad = _E_PAD - _E
    src3 = jnp.concatenate(
        [edge_index[0], jnp.zeros((pad,), jnp.int32)]).reshape(-1, _CHUNK_G,
                                                               128)
    dst3 = jnp.concatenate(
        [edge_index[1], jnp.zeros((pad,), jnp.int32)]).reshape(-1, _CHUNK_G,
                                                               128)
    w3 = lax.bitcast_convert_type(
        jnp.concatenate([edge_weight, jnp.zeros((pad,), jnp.float32)]),
        jnp.int32).reshape(-1, _CHUNK_G, 128)
    meta = jnp.stack([src3, dst3, w3], axis=1)
    emb0 = jnp.concatenate([user_emb_w, item_emb_w], axis=0)
    emb0 = jnp.concatenate(
        [emb0, jnp.zeros((_PAD_NODES - _N_NODES, _D), jnp.float32)], axis=0)
    emb = jnp.stack([emb0[:, :_DH], emb0[:, _DH:]], axis=0)
    acc = emb
    for layer in range(3):
        emb, acc = _make_layer(0.25 if layer == 2 else 1.0)(
            emb, acc, meta)
    final = jnp.concatenate([acc[0, :_N_NODES], acc[1, :_N_NODES]], axis=1)
    return final[:_N_USERS], final[_N_USERS:]


# speedup vs baseline: 14.2968x; 1.0704x over previous
import functools

import jax
import jax.numpy as jnp
import numpy as np
from jax import lax
from jax.experimental import pallas as pl
from jax.experimental.pallas import tpu as pltpu
from jax.experimental.pallas import tpu_sc as plsc

_N_USERS = 50000
_N_ITEMS = 50000
_N_NODES = _N_USERS + _N_ITEMS
_D = 32
_E = 1600000

_NC = 2
_NS = 16
_L = 16
_DH = _D // _NC

_PAD_NODES = 100352
_RPT = _PAD_NODES // _NS
_CHUNK_G = 4
_CHUNK = _CHUNK_G * 128
_NCHUNKS = 198
_EPT = _CHUNK * _NCHUNKS
_E_PAD = _EPT * _NS
_WB = 448


def _edge_pass(emb_in, meta, acc, idxq, rowsq, sem_i, sem_g, sem_s, c, cg0):
    def fire_idx(t, q):
        pltpu.async_copy(meta.at[cg0 + t], idxq.at[q], sem_i.at[q])

    def fire_gathers(t, q):
        off = q * np.int32(_CHUNK)
        for g in range(_CHUNK_G):
            pltpu.async_copy(
                emb_in.at[c].at[idxq.at[q, 0, g]],
                rowsq.at[pl.ds(off + np.int32(g * 128), 128)],
                sem_g.at[q])

    fire_idx(np.int32(0), np.int32(0))
    fire_idx(np.int32(1), np.int32(1))
    pltpu.make_async_copy(meta.at[0], idxq.at[0], sem_i.at[0]).wait()
    fire_gathers(np.int32(0), np.int32(0))

    @pl.loop(np.int32(0), np.int32(_NCHUNKS))
    def _chunk(t):
        p = t & np.int32(1)
        pn = np.int32(1) - p
        poff = p * np.int32(_CHUNK)

        @pl.when(t <= np.int32(_NCHUNKS - 2))
        def _fire_next():
            pltpu.make_async_copy(meta.at[0], idxq.at[pn],
                                  sem_i.at[pn]).wait()
            fire_gathers(t + np.int32(1), pn)

        pltpu.make_async_copy(emb_in.at[c].at[pl.ds(0, _CHUNK)],
                              rowsq.at[pl.ds(poff, _CHUNK)],
                              sem_g.at[p]).wait()

        @plsc.parallel_loop(np.int32(0), np.int32(_CHUNK // _L), unroll=2)
        def _scale_rows(q):
            g = q >> np.int32(3)
            h = q & np.int32(7)
            w16 = plsc.bitcast(idxq[p, 2, g, pl.ds(h * np.int32(_L), _L)],
                               jnp.float32)
            e0 = poff + q * np.int32(_L)
            for j in range(_L):
                e = e0 + np.int32(j)
                wspl = jnp.full((_L,), w16[j], jnp.float32)
                rowsq[e] = rowsq[e] * wspl

        descs = [pltpu.async_copy(
                     rowsq.at[pl.ds(poff + np.int32(g * 128), 128)],
                     acc.at[idxq.at[p, 1, g]], sem_s, add=True)
                 for g in range(_CHUNK_G)]
        for d in descs:
            d.wait()

        @pl.when(t <= np.int32(_NCHUNKS - 3))
        def _prefetch_idx():
            fire_idx(t + np.int32(2), p)


def _body(emb0, meta, sum_out, emba, embb,
          acc, idxq, rowsq, sem_i, sem_g, sem_s):
    c = lax.axis_index("c")
    s = lax.axis_index("s")
    zero = jnp.zeros((_L,), jnp.float32)
    cg0 = s * np.int32(_NCHUNKS)

    for layer in range(3):
        emb_in = (emb0, emba, embb)[layer]
        emb_out = (emba, embb, None)[layer]
        sum_src = emb0 if layer == 0 else sum_out
        scale = 0.25 if layer == 2 else 1.0

        @pl.loop(np.int32(0), np.int32(_WB))
        def _zero_rows(r):
            rows_z = rowsq.at[r]
            rows_z[...] = zero

        for k in range(_RPT // _WB):
            pltpu.sync_copy(
                rowsq.at[pl.ds(0, _WB)],
                acc.at[pl.ds(s * np.int32(_RPT) + np.int32(k * _WB), _WB)])
        plsc.subcore_barrier()

        _edge_pass(emb_in, meta, acc, idxq, rowsq,
                   sem_i, sem_g, sem_s, c, cg0)
        plsc.subcore_barrier()

        for k in range(_RPT // _WB):
            lrow = s * np.int32(_RPT) + np.int32(k * _WB)
            pltpu.sync_copy(acc.at[pl.ds(lrow, _WB)], rowsq.at[pl.ds(0, _WB)])
            pltpu.sync_copy(sum_src.at[c].at[pl.ds(lrow, _WB)],
                            rowsq.at[pl.ds(_CHUNK, _WB)])

            @pl.loop(np.int32(0), np.int32(_WB))
            def _accumulate(r):
                v = rowsq[r] + rowsq[r + np.int32(_CHUNK)]
                if scale != 1.0:
                    v = v * np.float32(scale)
                rowsq[r + np.int32(_CHUNK)] = v

            if emb_out is not None:
                pltpu.sync_copy(rowsq.at[pl.ds(0, _WB)],
                                emb_out.at[c].at[pl.ds(lrow, _WB)])
            pltpu.sync_copy(rowsq.at[pl.ds(_CHUNK, _WB)],
                            sum_out.at[c].at[pl.ds(lrow, _WB)])
        plsc.subcore_barrier()


@functools.lru_cache(maxsize=None)
def _make_kernel():
    mesh = plsc.VectorSubcoreMesh(
        core_axis_name="c", subcore_axis_name="s",
        num_cores=_NC, num_subcores=_NS)
    tbl = jax.ShapeDtypeStruct((_NC, _PAD_NODES, _DH), jnp.float32)
    scratch = [
        pltpu.VMEM_SHARED((_PAD_NODES, _DH), jnp.float32),
        pltpu.VMEM((2, 3, _CHUNK_G, 128), jnp.int32),
        pltpu.VMEM((2 * _CHUNK, _DH), jnp.float32),
        pltpu.SemaphoreType.DMA((2,)),
        pltpu.SemaphoreType.DMA((2,)),
        pltpu.SemaphoreType.DMA,
    ]
    return pl.kernel(_body, out_type=(tbl, tbl, tbl), mesh=mesh,
                     scratch_types=scratch,
                     compiler_params=pltpu.CompilerParams(
                         use_tc_tiling_on_sc=False,
                         needs_layout_passes=False))


def kernel(edge_index, edge_weight, user_emb_w, item_emb_w):
    with jax.enable_x64(False):
        return _kernel_x32(edge_index.astype(jnp.int32),
                           edge_weight.astype(jnp.float32),
                           user_emb_w.astype(jnp.float32),
                           item_emb_w.astype(jnp.float32))


def _kernel_x32(edge_index, edge_weight, user_emb_w, item_emb_w):
    pad = _E_PAD - _E
    src3 = jnp.concatenate(
        [edge_index[0], jnp.zeros((pad,), jnp.int32)]).reshape(-1, _CHUNK_G,
                                                               128)
    dst3 = jnp.concatenate(
        [edge_index[1], jnp.zeros((pad,), jnp.int32)]).reshape(-1, _CHUNK_G,
                                                               128)
    w3 = lax.bitcast_convert_type(
        jnp.concatenate([edge_weight, jnp.zeros((pad,), jnp.float32)]),
        jnp.int32).reshape(-1, _CHUNK_G, 128)
    meta = jnp.stack([src3, dst3, w3], axis=1)
    emb0 = jnp.concatenate([user_emb_w, item_emb_w], axis=0)
    emb0 = jnp.concatenate(
        [emb0, jnp.zeros((_PAD_NODES - _N_NODES, _D), jnp.float32)], axis=0)
    emb = jnp.stack([emb0[:, :_DH], emb0[:, _DH:]], axis=0)
    acc, _, _ = _make_kernel()(emb, meta)
    final = jnp.concatenate([acc[0, :_N_NODES], acc[1, :_N_NODES]], axis=1)
    return final[:_N_USERS], final[_N_USERS:]


# speedup vs baseline: 14.4651x; 1.0118x over previous
import functools

import jax
import jax.numpy as jnp
import numpy as np
from jax import lax
from jax.experimental import pallas as pl
from jax.experimental.pallas import tpu as pltpu
from jax.experimental.pallas import tpu_sc as plsc

_N_USERS = 50000
_N_ITEMS = 50000
_N_NODES = _N_USERS + _N_ITEMS
_D = 32
_E = 1600000

_NC = 2
_NS = 16
_L = 16
_DH = _D // _NC

_PAD_NODES = 100352
_RPT = _PAD_NODES // _NS
_CHUNK_G = 4
_CHUNK = _CHUNK_G * 128
_NCHUNKS = 198
_EPT = _CHUNK * _NCHUNKS
_E_PAD = _EPT * _NS
_WB = 448


def _edge_pass(emb_in, meta, acc, idxq, rowsq, sem_i, sem_g, sem_s, c, cg0):
    def fire_idx(t, q):
        pltpu.async_copy(meta.at[cg0 + t], idxq.at[q], sem_i.at[q])

    def fire_gathers(t, q):
        off = q * np.int32(_CHUNK)
        for g in range(_CHUNK_G):
            pltpu.async_copy(
                emb_in.at[c].at[idxq.at[q, 0, g]],
                rowsq.at[pl.ds(off + np.int32(g * 128), 128)],
                sem_g.at[q])

    fire_idx(np.int32(0), np.int32(0))
    fire_idx(np.int32(1), np.int32(1))
    pltpu.make_async_copy(meta.at[0], idxq.at[0], sem_i.at[0]).wait()
    fire_gathers(np.int32(0), np.int32(0))

    @pl.loop(np.int32(0), np.int32(_NCHUNKS))
    def _chunk(t):
        p = t & np.int32(1)
        pn = np.int32(1) - p
        poff = p * np.int32(_CHUNK)

        @pl.when(t <= np.int32(_NCHUNKS - 2))
        def _fire_next():
            pltpu.make_async_copy(meta.at[0], idxq.at[pn],
                                  sem_i.at[pn]).wait()
            fire_gathers(t + np.int32(1), pn)

        pltpu.make_async_copy(emb_in.at[c].at[pl.ds(0, _CHUNK)],
                              rowsq.at[pl.ds(poff, _CHUNK)],
                              sem_g.at[p]).wait()

        @plsc.parallel_loop(np.int32(0), np.int32(_CHUNK // _L), unroll=2)
        def _scale_rows(q):
            g = q >> np.int32(3)
            h = q & np.int32(7)
            w16 = plsc.bitcast(idxq[p, 2, g, pl.ds(h * np.int32(_L), _L)],
                               jnp.float32)
            e0 = poff + q * np.int32(_L)
            for j in range(_L):
                e = e0 + np.int32(j)
                wspl = jnp.full((_L,), w16[j], jnp.float32)
                rowsq[e] = rowsq[e] * wspl

        descs = [pltpu.async_copy(
                     rowsq.at[pl.ds(poff + np.int32(g * 128), 128)],
                     acc.at[idxq.at[p, 1, g]], sem_s, add=True)
                 for g in range(_CHUNK_G)]
        for d in descs:
            d.wait()

        @pl.when(t <= np.int32(_NCHUNKS - 3))
        def _prefetch_idx():
            fire_idx(t + np.int32(2), p)


def _body(emb0, meta, sum_out,
          acc, emba, embb, idxq, rowsq, sem_i, sem_g, sem_s):
    c = lax.axis_index("c")
    s = lax.axis_index("s")
    zero = jnp.zeros((_L,), jnp.float32)
    cg0 = s * np.int32(_NCHUNKS)

    for layer in range(3):
        emb_in = (emb0, emba, embb)[layer]
        emb_out = (emba, embb, None)[layer]
        sum_src = emb0 if layer == 0 else sum_out
        scale = 0.25 if layer == 2 else 1.0

        @pl.loop(np.int32(0), np.int32(_WB))
        def _zero_rows(r):
            rows_z = rowsq.at[r]
            rows_z[...] = zero

        for k in range(_RPT // _WB):
            pltpu.sync_copy(
                rowsq.at[pl.ds(0, _WB)],
                acc.at[pl.ds(s * np.int32(_RPT) + np.int32(k * _WB), _WB)])
        plsc.subcore_barrier()

        _edge_pass(emb_in, meta, acc, idxq, rowsq,
                   sem_i, sem_g, sem_s, c, cg0)
        plsc.subcore_barrier()

        for k in range(_RPT // _WB):
            lrow = s * np.int32(_RPT) + np.int32(k * _WB)
            pltpu.sync_copy(acc.at[pl.ds(lrow, _WB)], rowsq.at[pl.ds(0, _WB)])
            pltpu.sync_copy(sum_src.at[c].at[pl.ds(lrow, _WB)],
                            rowsq.at[pl.ds(_CHUNK, _WB)])

            @pl.loop(np.int32(0), np.int32(_WB))
            def _accumulate(r):
                v = rowsq[r] + rowsq[r + np.int32(_CHUNK)]
                if scale != 1.0:
                    v = v * np.float32(scale)
                rowsq[r + np.int32(_CHUNK)] = v

            if emb_out is not None:
                pltpu.sync_copy(rowsq.at[pl.ds(0, _WB)],
                                emb_out.at[c].at[pl.ds(lrow, _WB)])
            pltpu.sync_copy(rowsq.at[pl.ds(_CHUNK, _WB)],
                            sum_out.at[c].at[pl.ds(lrow, _WB)])
        plsc.subcore_barrier()


@functools.lru_cache(maxsize=None)
def _make_kernel():
    mesh = plsc.VectorSubcoreMesh(
        core_axis_name="c", subcore_axis_name="s",
        num_cores=_NC, num_subcores=_NS)
    tbl = jax.ShapeDtypeStruct((_NC, _PAD_NODES, _DH), jnp.float32)
    scratch = [
        pltpu.VMEM_SHARED((_PAD_NODES, _DH), jnp.float32),
        pltpu.HBM((_NC, _PAD_NODES, _DH), jnp.float32),
        pltpu.HBM((_NC, _PAD_NODES, _DH), jnp.float32),
        pltpu.VMEM((2, 3, _CHUNK_G, 128), jnp.int32),
        pltpu.VMEM((2 * _CHUNK, _DH), jnp.float32),
        pltpu.SemaphoreType.DMA((2,)),
        pltpu.SemaphoreType.DMA((2,)),
        pltpu.SemaphoreType.DMA,
    ]
    return pl.kernel(_body, out_type=tbl, mesh=mesh,
                     scratch_types=scratch,
                     compiler_params=pltpu.CompilerParams(
                         use_tc_tiling_on_sc=False,
                         needs_layout_passes=False))


def kernel(edge_index, edge_weight, user_emb_w, item_emb_w):
    with jax.enable_x64(False):
        return _kernel_x32(edge_index.astype(jnp.int32),
                           edge_weight.astype(jnp.float32),
                           user_emb_w.astype(jnp.float32),
                           item_emb_w.astype(jnp.float32))


def _kernel_x32(edge_index, edge_weight, user_emb_w, item_emb_w):
    pad = _E_PAD - _E
    src3 = jnp.concatenate(
        [edge_index[0], jnp.zeros((pad,), jnp.int32)]).reshape(-1, _CHUNK_G,
                                                               128)
    dst3 = jnp.concatenate(
        [edge_index[1], jnp.zeros((pad,), jnp.int32)]).reshape(-1, _CHUNK_G,
                                                               128)
    w3 = lax.bitcast_convert_type(
        jnp.concatenate([edge_weight, jnp.zeros((pad,), jnp.float32)]),
        jnp.int32).reshape(-1, _CHUNK_G, 128)
    meta = jnp.stack([src3, dst3, w3], axis=1)
    emb0 = jnp.concatenate([user_emb_w, item_emb_w], axis=0)
    emb0 = jnp.concatenate(
        [emb0, jnp.zeros((_PAD_NODES - _N_NODES, _D), jnp.float32)], axis=0)
    emb = jnp.stack([emb0[:, :_DH], emb0[:, _DH:]], axis=0)
    acc = _make_kernel()(emb, meta)
    final = jnp.concatenate([acc[0, :_N_NODES], acc[1, :_N_NODES]], axis=1)
    return final[:_N_USERS], final[_N_USERS:]


# speedup vs baseline: 16.5629x; 1.1450x over previous
import functools

import jax
import jax.numpy as jnp
import numpy as np
from jax import lax
from jax.experimental import pallas as pl
from jax.experimental.pallas import tpu as pltpu
from jax.experimental.pallas import tpu_sc as plsc

_N_USERS = 50000
_N_ITEMS = 50000
_N_NODES = _N_USERS + _N_ITEMS
_D = 32
_E = 1600000

_NC = 2
_NS = 16
_L = 16
_DH = _D // _NC

_PAD_NODES = 100352
_RPT = _PAD_NODES // _NS
_CHUNK_G = 4
_CHUNK = _CHUNK_G * 128
_NCHUNKS = 198
_EPT = _CHUNK * _NCHUNKS
_E_PAD = _EPT * _NS
_WB = 448


def _edge_pass(emb_in, meta, acc, idxq, rowsq, sem_i, sem_g, sem_s, c, cg0):
    def fire_idx(t, q):
        pltpu.async_copy(meta.at[cg0 + t], idxq.at[q], sem_i.at[q])

    def fire_gathers(t, q):
        ph = t & np.int32(1)
        off = ph * np.int32(_CHUNK)
        for g in range(_CHUNK_G):
            pltpu.async_copy(
                emb_in.at[c].at[idxq.at[q, 0, g]],
                rowsq.at[pl.ds(off + np.int32(g * 128), 128)],
                sem_g.at[ph])

    fire_idx(np.int32(0), np.int32(0))
    fire_idx(np.int32(1), np.int32(1))
    pltpu.make_async_copy(meta.at[0], idxq.at[0], sem_i.at[0]).wait()
    fire_gathers(np.int32(0), np.int32(0))

    def wait_scatters(ph, slot):
        off = ph * np.int32(_CHUNK)
        for g in range(_CHUNK_G):
            pltpu.make_async_copy(
                rowsq.at[pl.ds(off + np.int32(g * 128), 128)],
                acc.at[idxq.at[slot, 1, g]], sem_s.at[ph]).wait()

    @pl.loop(np.int32(0), np.int32(_NCHUNKS), init_carry=np.int32(0))
    def _chunk(t, m):
        p = t & np.int32(1)
        pn = np.int32(1) - p
        poff = p * np.int32(_CHUNK)
        mn = m + np.int32(1)
        mn = jnp.where(mn == np.int32(3), np.int32(0), mn)
        m2 = mn + np.int32(1)
        m2 = jnp.where(m2 == np.int32(3), np.int32(0), m2)

        @pl.when(t >= np.int32(1))
        def _drain_prev_scatters():
            wait_scatters(pn, m2)

        @pl.when(t <= np.int32(_NCHUNKS - 2))
        def _fire_next():
            pltpu.make_async_copy(meta.at[0], idxq.at[mn],
                                  sem_i.at[mn]).wait()
            fire_gathers(t + np.int32(1), mn)

        @pl.when(t <= np.int32(_NCHUNKS - 3))
        def _prefetch_idx():
            fire_idx(t + np.int32(2), m2)

        pltpu.make_async_copy(emb_in.at[c].at[pl.ds(0, _CHUNK)],
                              rowsq.at[pl.ds(poff, _CHUNK)],
                              sem_g.at[p]).wait()

        @plsc.parallel_loop(np.int32(0), np.int32(_CHUNK // _L), unroll=2)
        def _scale_rows(q):
            g = q >> np.int32(3)
            h = q & np.int32(7)
            w16 = plsc.bitcast(idxq[m, 2, g, pl.ds(h * np.int32(_L), _L)],
                               jnp.float32)
            e0 = poff + q * np.int32(_L)
            for j in range(_L):
                e = e0 + np.int32(j)
                wspl = jnp.full((_L,), w16[j], jnp.float32)
                rowsq[e] = rowsq[e] * wspl

        for g in range(_CHUNK_G):
            pltpu.async_copy(
                rowsq.at[pl.ds(poff + np.int32(g * 128), 128)],
                acc.at[idxq.at[m, 1, g]], sem_s.at[p], add=True)
        return mn

    wait_scatters(np.int32((_NCHUNKS - 1) & 1), np.int32((_NCHUNKS - 1) % 3))


def _body(emb0, meta, sum_out,
          acc, emba, embb, idxq, rowsq, sem_i, sem_g, sem_s):
    c = lax.axis_index("c")
    s = lax.axis_index("s")
    zero = jnp.zeros((_L,), jnp.float32)
    cg0 = s * np.int32(_NCHUNKS)

    for layer in range(3):
        emb_in = (emb0, emba, embb)[layer]
        emb_out = (emba, embb, None)[layer]
        sum_src = emb0 if layer == 0 else sum_out
        scale = 0.25 if layer == 2 else 1.0

        @pl.loop(np.int32(0), np.int32(_WB))
        def _zero_rows(r):
            rows_z = rowsq.at[r]
            rows_z[...] = zero

        for k in range(_RPT // _WB):
            pltpu.sync_copy(
                rowsq.at[pl.ds(0, _WB)],
                acc.at[pl.ds(s * np.int32(_RPT) + np.int32(k * _WB), _WB)])
        plsc.subcore_barrier()

        _edge_pass(emb_in, meta, acc, idxq, rowsq,
                   sem_i, sem_g, sem_s, c, cg0)
        plsc.subcore_barrier()

        for k in range(_RPT // _WB):
            lrow = s * np.int32(_RPT) + np.int32(k * _WB)
            pltpu.sync_copy(acc.at[pl.ds(lrow, _WB)], rowsq.at[pl.ds(0, _WB)])
            pltpu.sync_copy(sum_src.at[c].at[pl.ds(lrow, _WB)],
                            rowsq.at[pl.ds(_CHUNK, _WB)])

            @pl.loop(np.int32(0), np.int32(_WB))
            def _accumulate(r):
                v = rowsq[r] + rowsq[r + np.int32(_CHUNK)]
                if scale != 1.0:
                    v = v * np.float32(scale)
                rowsq[r + np.int32(_CHUNK)] = v

            if emb_out is not None:
                pltpu.sync_copy(rowsq.at[pl.ds(0, _WB)],
                                emb_out.at[c].at[pl.ds(lrow, _WB)])
            pltpu.sync_copy(rowsq.at[pl.ds(_CHUNK, _WB)],
                            sum_out.at[c].at[pl.ds(lrow, _WB)])
        plsc.subcore_barrier()


@functools.lru_cache(maxsize=None)
def _make_kernel():
    mesh = plsc.VectorSubcoreMesh(
        core_axis_name="c", subcore_axis_name="s",
        num_cores=_NC, num_subcores=_NS)
    tbl = jax.ShapeDtypeStruct((_NC, _PAD_NODES, _DH), jnp.float32)
    scratch = [
        pltpu.VMEM_SHARED((_PAD_NODES, _DH), jnp.float32),
        pltpu.HBM((_NC, _PAD_NODES, _DH), jnp.float32),
        pltpu.HBM((_NC, _PAD_NODES, _DH), jnp.float32),
        pltpu.VMEM((3, 3, _CHUNK_G, 128), jnp.int32),
        pltpu.VMEM((2 * _CHUNK, _DH), jnp.float32),
        pltpu.SemaphoreType.DMA((3,)),
        pltpu.SemaphoreType.DMA((2,)),
        pltpu.SemaphoreType.DMA((2,)),
    ]
    return pl.kernel(_body, out_type=tbl, mesh=mesh,
                     scratch_types=scratch,
                     compiler_params=pltpu.CompilerParams(
                         use_tc_tiling_on_sc=False,
                         needs_layout_passes=False))


def kernel(edge_index, edge_weight, user_emb_w, item_emb_w):
    with jax.enable_x64(False):
        return _kernel_x32(edge_index.astype(jnp.int32),
                           edge_weight.astype(jnp.float32),
                           user_emb_w.astype(jnp.float32),
                           item_emb_w.astype(jnp.float32))


def _kernel_x32(edge_index, edge_weight, user_emb_w, item_emb_w):
    pad = _E_PAD - _E
    src3 = jnp.concatenate(
        [edge_index[0], jnp.zeros((pad,), jnp.int32)]).reshape(-1, _CHUNK_G,
                                                               128)
    dst3 = jnp.concatenate(
        [edge_index[1], jnp.zeros((pad,), jnp.int32)]).reshape(-1, _CHUNK_G,
                                                               128)
    w3 = lax.bitcast_convert_type(
        jnp.concatenate([edge_weight, jnp.zeros((pad,), jnp.float32)]),
        jnp.int32).reshape(-1, _CHUNK_G, 128)
    meta = jnp.stack([src3, dst3, w3], axis=1)
    emb0 = jnp.concatenate([user_emb_w, item_emb_w], axis=0)
    emb0 = jnp.concatenate(
        [emb0, jnp.zeros((_PAD_NODES - _N_NODES, _D), jnp.float32)], axis=0)
    emb = jnp.stack([emb0[:, :_DH], emb0[:, _DH:]], axis=0)
    acc = _make_kernel()(emb, meta)
    final = jnp.concatenate([acc[0, :_N_NODES], acc[1, :_N_NODES]], axis=1)
    return final[:_N_USERS], final[_N_USERS:]


# speedup vs baseline: 16.9306x; 1.0222x over previous
import functools

import jax
import jax.numpy as jnp
import numpy as np
from jax import lax
from jax.experimental import pallas as pl
from jax.experimental.pallas import tpu as pltpu
from jax.experimental.pallas import tpu_sc as plsc

_N_USERS = 50000
_N_ITEMS = 50000
_N_NODES = _N_USERS + _N_ITEMS
_D = 32
_E = 1600000

_NC = 2
_NS = 16
_L = 16
_DH = _D // _NC

_PAD_NODES = 100352
_RPT = _PAD_NODES // _NS
_CHUNK_G = 4
_CHUNK = _CHUNK_G * 128
_NCHUNKS = 198
_EPT = _CHUNK * _NCHUNKS
_E_PAD = _EPT * _NS
_WB = 448
_WBP = 224
_NWB = _RPT // _WBP


def _edge_pass(emb_in, meta, acc, idxq, rowsq, sem_i, sem_g, sem_s, c, cg0):
    def fire_idx(t, q):
        pltpu.async_copy(meta.at[cg0 + t], idxq.at[q], sem_i.at[q])

    def fire_gathers(t, q):
        ph = t & np.int32(1)
        off = ph * np.int32(_CHUNK)
        for g in range(_CHUNK_G):
            pltpu.async_copy(
                emb_in.at[c].at[idxq.at[q, 0, g]],
                rowsq.at[pl.ds(off + np.int32(g * 128), 128)],
                sem_g.at[ph])

    fire_idx(np.int32(0), np.int32(0))
    fire_idx(np.int32(1), np.int32(1))
    pltpu.make_async_copy(meta.at[0], idxq.at[0], sem_i.at[0]).wait()
    fire_gathers(np.int32(0), np.int32(0))

    def wait_scatters(ph, slot):
        off = ph * np.int32(_CHUNK)
        for g in range(_CHUNK_G):
            pltpu.make_async_copy(
                rowsq.at[pl.ds(off + np.int32(g * 128), 128)],
                acc.at[idxq.at[slot, 1, g]], sem_s.at[ph]).wait()

    @pl.loop(np.int32(0), np.int32(_NCHUNKS), init_carry=np.int32(0))
    def _chunk(t, m):
        p = t & np.int32(1)
        pn = np.int32(1) - p
        poff = p * np.int32(_CHUNK)
        mn = m + np.int32(1)
        mn = jnp.where(mn == np.int32(3), np.int32(0), mn)
        m2 = mn + np.int32(1)
        m2 = jnp.where(m2 == np.int32(3), np.int32(0), m2)

        @pl.when(t >= np.int32(1))
        def _drain_prev_scatters():
            wait_scatters(pn, m2)

        @pl.when(t <= np.int32(_NCHUNKS - 2))
        def _fire_next():
            pltpu.make_async_copy(meta.at[0], idxq.at[mn],
                                  sem_i.at[mn]).wait()
            fire_gathers(t + np.int32(1), mn)

        @pl.when(t <= np.int32(_NCHUNKS - 3))
        def _prefetch_idx():
            fire_idx(t + np.int32(2), m2)

        pltpu.make_async_copy(emb_in.at[c].at[pl.ds(0, _CHUNK)],
                              rowsq.at[pl.ds(poff, _CHUNK)],
                              sem_g.at[p]).wait()

        @plsc.parallel_loop(np.int32(0), np.int32(_CHUNK // _L), unroll=2)
        def _scale_rows(q):
            g = q >> np.int32(3)
            h = q & np.int32(7)
            w16 = plsc.bitcast(idxq[m, 2, g, pl.ds(h * np.int32(_L), _L)],
                               jnp.float32)
            e0 = poff + q * np.int32(_L)
            for j in range(_L):
                e = e0 + np.int32(j)
                wspl = jnp.full((_L,), w16[j], jnp.float32)
                rowsq[e] = rowsq[e] * wspl

        for g in range(_CHUNK_G):
            pltpu.async_copy(
                rowsq.at[pl.ds(poff + np.int32(g * 128), 128)],
                acc.at[idxq.at[m, 1, g]], sem_s.at[p], add=True)
        return mn

    wait_scatters(np.int32((_NCHUNKS - 1) & 1), np.int32((_NCHUNKS - 1) % 3))


def _body(emb0, meta, sum_out,
          acc, emba, embb, idxq, rowsq, sem_i, sem_g, sem_s):
    c = lax.axis_index("c")
    s = lax.axis_index("s")
    zero = jnp.zeros((_L,), jnp.float32)
    cg0 = s * np.int32(_NCHUNKS)

    for layer in range(3):
        emb_in = (emb0, emba, embb)[layer]
        emb_out = (emba, embb, None)[layer]
        sum_src = emb0 if layer == 0 else sum_out
        scale = 0.25 if layer == 2 else 1.0

        @pl.loop(np.int32(0), np.int32(_WB))
        def _zero_rows(r):
            rows_z = rowsq.at[r]
            rows_z[...] = zero

        for k in range(_RPT // _WB):
            pltpu.sync_copy(
                rowsq.at[pl.ds(0, _WB)],
                acc.at[pl.ds(s * np.int32(_RPT) + np.int32(k * _WB), _WB)])
        plsc.subcore_barrier()

        _edge_pass(emb_in, meta, acc, idxq, rowsq,
                   sem_i, sem_g, sem_s, c, cg0)
        plsc.subcore_barrier()

        row0 = s * np.int32(_RPT)
        nst = 448 if emb_out is not None else 224

        def fire_loads(k):
            kp = k & np.int32(1)
            off = kp * np.int32(512)
            lrow = row0 + k * np.int32(_WBP)
            pltpu.async_copy(acc.at[pl.ds(lrow, _WBP)],
                             rowsq.at[pl.ds(off, _WBP)], sem_g.at[kp])
            pltpu.async_copy(sum_src.at[c].at[pl.ds(lrow, _WBP)],
                             rowsq.at[pl.ds(off + np.int32(_WBP), _WBP)],
                             sem_i.at[kp])

        def drain_stores(kp):
            pltpu.make_async_copy(
                rowsq.at[pl.ds(kp * np.int32(512), nst)],
                sum_out.at[c].at[pl.ds(row0, nst)],
                sem_s.at[kp]).wait()

        fire_loads(np.int32(0))

        @pl.loop(np.int32(0), np.int32(_NWB))
        def _writeback(k):
            kp = k & np.int32(1)
            kpn = np.int32(1) - kp
            off = kp * np.int32(512)
            lrow = row0 + k * np.int32(_WBP)

            @pl.when(k >= np.int32(1))
            def _drain_prev_stores():
                drain_stores(kpn)

            @pl.when(k <= np.int32(_NWB - 2))
            def _fire_next_loads():
                fire_loads(k + np.int32(1))

            pltpu.make_async_copy(acc.at[pl.ds(lrow, _WBP)],
                                  rowsq.at[pl.ds(off, _WBP)],
                                  sem_g.at[kp]).wait()
            pltpu.make_async_copy(sum_src.at[c].at[pl.ds(lrow, _WBP)],
                                  rowsq.at[pl.ds(off + np.int32(_WBP), _WBP)],
                                  sem_i.at[kp]).wait()

            @plsc.parallel_loop(np.int32(0), np.int32(_WBP), unroll=4)
            def _accumulate(r):
                v = (rowsq[off + r] + rowsq[off + r + np.int32(_WBP)])
                if scale != 1.0:
                    v = v * np.float32(scale)
                rowsq[off + r + np.int32(_WBP)] = v

            if emb_out is not None:
                pltpu.async_copy(rowsq.at[pl.ds(off, _WBP)],
                                 emb_out.at[c].at[pl.ds(lrow, _WBP)],
                                 sem_s.at[kp])
            pltpu.async_copy(rowsq.at[pl.ds(off + np.int32(_WBP), _WBP)],
                             sum_out.at[c].at[pl.ds(lrow, _WBP)],
                             sem_s.at[kp])

        drain_stores(np.int32((_NWB - 1) & 1))
        plsc.subcore_barrier()


@functools.lru_cache(maxsize=None)
def _make_kernel():
    mesh = plsc.VectorSubcoreMesh(
        core_axis_name="c", subcore_axis_name="s",
        num_cores=_NC, num_subcores=_NS)
    tbl = jax.ShapeDtypeStruct((_NC, _PAD_NODES, _DH), jnp.float32)
    scratch = [
        pltpu.VMEM_SHARED((_PAD_NODES, _DH), jnp.float32),
        pltpu.HBM((_NC, _PAD_NODES, _DH), jnp.float32),
        pltpu.HBM((_NC, _PAD_NODES, _DH), jnp.float32),
        pltpu.VMEM((3, 3, _CHUNK_G, 128), jnp.int32),
        pltpu.VMEM((2 * _CHUNK, _DH), jnp.float32),
        pltpu.SemaphoreType.DMA((3,)),
        pltpu.SemaphoreType.DMA((2,)),
        pltpu.SemaphoreType.DMA((2,)),
    ]
    return pl.kernel(_body, out_type=tbl, mesh=mesh,
                     scratch_types=scratch,
                     compiler_params=pltpu.CompilerParams(
                         use_tc_tiling_on_sc=False,
                         needs_layout_passes=False))


def kernel(edge_index, edge_weight, user_emb_w, item_emb_w):
    with jax.enable_x64(False):
        return _kernel_x32(edge_index.astype(jnp.int32),
                           edge_weight.astype(jnp.float32),
                           user_emb_w.astype(jnp.float32),
                           item_emb_w.astype(jnp.float32))


def _kernel_x32(edge_index, edge_weight, user_emb_w, item_emb_w):
    pad = _E_PAD - _E
    src3 = jnp.concatenate(
        [edge_index[0], jnp.zeros((pad,), jnp.int32)]).reshape(-1, _CHUNK_G,
                                                               128)
    dst3 = jnp.concatenate(
        [edge_index[1], jnp.zeros((pad,), jnp.int32)]).reshape(-1, _CHUNK_G,
                                                               128)
    w3 = lax.bitcast_convert_type(
        jnp.concatenate([edge_weight, jnp.zeros((pad,), jnp.float32)]),
        jnp.int32).reshape(-1, _CHUNK_G, 128)
    meta = jnp.stack([src3, dst3, w3], axis=1)
    emb0 = jnp.concatenate([user_emb_w, item_emb_w], axis=0)
    emb0 = jnp.concatenate(
        [emb0, jnp.zeros((_PAD_NODES - _N_NODES, _D), jnp.float32)], axis=0)
    emb = jnp.stack([emb0[:, :_DH], emb0[:, _DH:]], axis=0)
    acc = _make_kernel()(emb, meta)
    final = jnp.concatenate([acc[0, :_N_NODES], acc[1, :_N_NODES]], axis=1)
    return final[:_N_USERS], final[_N_USERS:]
